# baseline (speedup 1.0000x reference)
import functools

import jax
import jax.numpy as jnp
from jax import lax
from jax.experimental import pallas as pl
from jax.experimental.pallas import tpu as pltpu
from jax.experimental.pallas import tpu_sc as plsc

B = 16384
NUM_REG = 4
NUM_CAT = 6
NUM_VAR = NUM_REG + NUM_CAT
VOCAB = 100000
D = 64

NC, NS, L = 2, 16, 16
NW = NC * NS
CPW = B // NW
T = 128
NCH = CPW // T


def _sc_body(ai_t, tbl, w_hbm, b_hbm, out, ait_v, w_v, b_v, idx_v, dst_v,
             gath_v, reg_v, sem_g, sem_s):
    wid = lax.axis_index("s") * NC + lax.axis_index("c")
    gbase = wid * CPW

    pltpu.sync_copy(ai_t.at[:, pl.ds(gbase, CPW)], ait_v)
    pltpu.sync_copy(w_hbm, w_v)
    pltpu.sync_copy(b_hbm, b_v)

    iota = lax.iota(jnp.int32, L)

    for c in range(NCH):
        t0 = c * T

        def _idx_body(o, _):
            off = o * L
            tok = (gbase + t0 + off) + iota
            for j in range(NUM_CAT):
                v = ait_v[NUM_REG + j, pl.ds(t0 + off, L)]
                idx_v[c * NUM_CAT + j, pl.ds(off, L)] = (
                    v.astype(jnp.int32) + j * VOCAB)
            for j in range(NUM_VAR):
                dst_v[c * NUM_VAR + j, pl.ds(off, L)] = tok * NUM_VAR + j
            return 0

        lax.fori_loop(0, T // L, _idx_body, 0)

    w_regs = [[w_v[i, pl.ds(ch * L, L)] for ch in range(D // L)]
              for i in range(NUM_REG)]
    b_regs = [[b_v[i, pl.ds(ch * L, L)] for ch in range(D // L)]
              for i in range(NUM_REG)]

    for c in range(NCH):
        t0 = c * T
        for j in range(NUM_CAT):
            pltpu.async_copy(tbl.at[idx_v.at[c * NUM_CAT + j]], gath_v,
                             sem_g).wait()
            pltpu.async_copy(gath_v, out.at[dst_v.at[c * NUM_VAR + NUM_REG + j]],
                             sem_s).wait()
        for i in range(NUM_REG):
            def _reg_body(o, _):
                xv = ait_v[i, pl.ds(t0 + o * L, L)]
                for l in range(L):
                    x = xv[l]
                    for ch in range(D // L):
                        reg_v[o * L + l, pl.ds(ch * L, L)] = (
                            x * w_regs[i][ch] + b_regs[i][ch])
                return 0

            lax.fori_loop(0, T // L, _reg_body, 0)
            pltpu.async_copy(reg_v, out.at[dst_v.at[c * NUM_VAR + i]],
                             sem_s).wait()


@jax.jit
def _run(ai_t, tbl_flat, W, b):
    mesh = plsc.VectorSubcoreMesh(core_axis_name="c", subcore_axis_name="s")
    f = functools.partial(
        pl.kernel,
        out_type=jax.ShapeDtypeStruct((B * NUM_VAR, D), jnp.float32),
        mesh=mesh,
        scratch_types=[
            pltpu.VMEM((NUM_VAR, CPW), jnp.float32),
            pltpu.VMEM((NUM_REG, D), jnp.float32),
            pltpu.VMEM((NUM_REG, D), jnp.float32),
            pltpu.VMEM((NCH * NUM_CAT, T), jnp.int32),
            pltpu.VMEM((NCH * NUM_VAR, T), jnp.int32),
            pltpu.VMEM((T, D), jnp.float32),
            pltpu.VMEM((T, D), jnp.float32),
            pltpu.SemaphoreType.DMA,
            pltpu.SemaphoreType.DMA,
        ],
        compiler_params=pltpu.CompilerParams(use_tc_tiling_on_sc=False),
    )(_sc_body)
    return f(ai_t, tbl_flat, W, b)


def kernel(all_inputs, tables, W, b):
    ai_t = all_inputs.T
    tbl_flat = tables.reshape(NUM_CAT * VOCAB, D)
    out_flat = _run(ai_t, tbl_flat, W, b)
    return out_flat.reshape(B, NUM_VAR, D)


# speedup vs baseline: 1.0068x; 1.0068x over previous
import functools

import jax
import jax.numpy as jnp
from jax import lax
from jax.experimental import pallas as pl
from jax.experimental.pallas import tpu as pltpu
from jax.experimental.pallas import tpu_sc as plsc

B = 16384
NUM_REG = 4
NUM_CAT = 6
NUM_VAR = NUM_REG + NUM_CAT
VOCAB = 100000
D = 64

NC, NS, L = 2, 16, 16
NW = NC * NS
CPW = B // NW
T = 64
NCH = CPW // T
IPT = 8


def _sc_body(ai, tbl, w_hbm, b_hbm, out3, ai_v, w_v, b_v, idx1, stage_v,
             sem_g0, sem_g1, sem_o0, sem_o1):
    wid = lax.axis_index("s") * NC + lax.axis_index("c")
    gbase = wid * CPW
    sem_g = [sem_g0, sem_g1]
    sem_o = [sem_o0, sem_o1]

    pltpu.sync_copy(ai.at[pl.ds(gbase, CPW)], ai_v)
    pltpu.sync_copy(w_hbm, w_v)
    pltpu.sync_copy(b_hbm, b_v)

    iota = lax.iota(jnp.int32, L)
    cols = [jnp.full((L,), j, jnp.int32) for j in range(NUM_VAR)]

    lane_j = jnp.minimum(iota & (IPT - 1), NUM_CAT - 1)
    lane_dt = iota >> 3

    def _idx_body(g, _):
        t = g * 2 + lane_dt
        val = plsc.load_gather(ai_v, [t, NUM_REG + lane_j])
        idx1[pl.ds(g * L, L)] = val.astype(jnp.int32) + lane_j * VOCAB
        return 0

    lax.fori_loop(0, CPW * IPT // L, _idx_body, 0)

    w_regs = [[w_v[i, pl.ds(ch * L, L)] for ch in range(D // L)]
              for i in range(NUM_REG)]
    b_regs = [[b_v[i, pl.ds(ch * L, L)] for ch in range(D // L)]
              for i in range(NUM_REG)]

    def fire_gathers(c, s):
        def _g_body(tl, _):
            pltpu.async_copy(
                tbl.at[idx1.at[pl.ds((c * T + tl) * IPT, NUM_CAT)]],
                stage_v.at[s, tl, pl.ds(NUM_REG, NUM_CAT)], sem_g[s])
            return 0

        lax.fori_loop(0, T, _g_body, 0)

    def drain_gathers(s):
        def _d_body(_, __):
            pltpu.make_async_copy(
                tbl.at[pl.ds(0, NUM_CAT)],
                stage_v.at[s, 0, pl.ds(NUM_REG, NUM_CAT)], sem_g[s]).wait()
            return 0

        lax.fori_loop(0, T, _d_body, 0)

    def reg_fill(c, s):
        t0 = c * T
        for i in range(NUM_REG):
            def _reg_body(o, _):
                xv = plsc.load_gather(ai_v, [t0 + o * L + iota, cols[i]])
                for l in range(L):
                    x = xv[l]
                    for ch in range(D // L):
                        stage_v[s, o * L + l, i, pl.ds(ch * L, L)] = (
                            x * w_regs[i][ch] + b_regs[i][ch])
                return 0

            lax.fori_loop(0, T // L, _reg_body, 0)

    def fire_out(c, s):
        return pltpu.async_copy(
            stage_v.at[s], out3.at[pl.ds(gbase + c * T, T)], sem_o[s])

    od = {}
    fire_gathers(0, 0)
    for c in range(NCH):
        s = c % 2
        if c + 1 < NCH:
            if c - 1 in od:
                od.pop(c - 1).wait()
            fire_gathers(c + 1, s ^ 1)
        drain_gathers(s)
        reg_fill(c, s)
        od[c] = fire_out(c, s)
    for c in sorted(od):
        od.pop(c).wait()


@jax.jit
def _run(ai, tbl_flat, W, b):
    mesh = plsc.VectorSubcoreMesh(core_axis_name="c", subcore_axis_name="s")
    f = functools.partial(
        pl.kernel,
        out_type=jax.ShapeDtypeStruct((B, NUM_VAR, D), jnp.float32),
        mesh=mesh,
        scratch_types=[
            pltpu.VMEM((CPW, NUM_VAR), jnp.float32),
            pltpu.VMEM((NUM_REG, D), jnp.float32),
            pltpu.VMEM((NUM_REG, D), jnp.float32),
            pltpu.VMEM((CPW * IPT,), jnp.int32),
            pltpu.VMEM((2, T, NUM_VAR, D), jnp.float32),
            pltpu.SemaphoreType.DMA,
            pltpu.SemaphoreType.DMA,
            pltpu.SemaphoreType.DMA,
            pltpu.SemaphoreType.DMA,
        ],
        compiler_params=pltpu.CompilerParams(use_tc_tiling_on_sc=False,
                                             needs_layout_passes=False),
    )(_sc_body)
    return f(ai, tbl_flat, W, b)


def kernel(all_inputs, tables, W, b):
    tbl_flat = tables.reshape(NUM_CAT * VOCAB, D)
    return _run(all_inputs, tbl_flat, W, b)


# speedup vs baseline: 1.6317x; 1.6207x over previous
import functools

import jax
import jax.numpy as jnp
from jax import lax
from jax.experimental import pallas as pl
from jax.experimental.pallas import tpu as pltpu
from jax.experimental.pallas import tpu_sc as plsc

B = 16384
NUM_REG = 4
NUM_CAT = 6
NUM_VAR = NUM_REG + NUM_CAT
VOCAB = 100000
D = 64

NC, NS, L = 2, 16, 16
NW = NC * NS
KPW = NUM_VAR * D // NW
TQ = 4096
NQ = B // TQ
GPQ = TQ // L


def _sc_body(aiT, tblT, w_hbm, b_hbm, outT, plane_v, chunk_v, idx_v, oh0,
             oh1, w_v, b_v, sem_o0, sem_o1):
    wid = lax.axis_index("s") * NC + lax.axis_index("c")
    iota = lax.iota(jnp.int32, L)

    pltpu.sync_copy(w_hbm, w_v)
    pltpu.sync_copy(b_hbm, b_v)

    ohs = [oh0, oh1]
    sems = [sem_o0, sem_o1]
    last = [None, None]

    for k in range(KPW):
        v = k // 2
        d = 32 * (k % 2) + wid
        if v < NUM_REG:
            dvec = iota * 0 + d
            ivec = iota * 0 + v
            wsp = plsc.load_gather(w_v, [ivec, dvec])
            bsp = plsc.load_gather(b_v, [ivec, dvec])
            for q in range(NQ):
                p = q % 2
                pltpu.sync_copy(aiT.at[v, pl.ds(q * TQ, TQ)], chunk_v)
                if last[p] is not None:
                    last[p].wait()

                def _fma_body(g, _):
                    x = chunk_v[pl.ds(g * L, L)]
                    ohs[p][pl.ds(g * L, L)] = x * wsp + bsp
                    return 0

                lax.fori_loop(0, GPQ, _fma_body, 0)
                last[p] = pltpu.async_copy(
                    ohs[p], outT.at[v, d, pl.ds(q * TQ, TQ)], sems[p])
        else:
            jc = v - NUM_REG
            pltpu.sync_copy(tblT.at[jc, d], plane_v)
            for q in range(NQ):
                p = q % 2
                pltpu.sync_copy(aiT.at[NUM_REG + jc, pl.ds(q * TQ, TQ)],
                                chunk_v)

                def _cvt_body(g, _):
                    idx_v[pl.ds(g * L, L)] = (
                        chunk_v[pl.ds(g * L, L)].astype(jnp.int32))
                    return 0

                lax.fori_loop(0, GPQ, _cvt_body, 0)
                if last[p] is not None:
                    last[p].wait()

                def _g_body(g, _):
                    iv = idx_v[pl.ds(g * L, L)]
                    ohs[p][pl.ds(g * L, L)] = plsc.load_gather(plane_v, [iv])
                    return 0

                lax.fori_loop(0, GPQ, _g_body, 0)
                last[p] = pltpu.async_copy(
                    ohs[p], outT.at[NUM_REG + jc, d, pl.ds(q * TQ, TQ)],
                    sems[p])
    for p in range(2):
        if last[p] is not None:
            last[p].wait()


@jax.jit
def _run(aiT, tblT, W, b):
    mesh = plsc.VectorSubcoreMesh(core_axis_name="c", subcore_axis_name="s")
    f = functools.partial(
        pl.kernel,
        out_type=jax.ShapeDtypeStruct((NUM_VAR, D, B), jnp.float32),
        mesh=mesh,
        scratch_types=[
            pltpu.VMEM((VOCAB,), jnp.float32),
            pltpu.VMEM((TQ,), jnp.float32),
            pltpu.VMEM((TQ,), jnp.int32),
            pltpu.VMEM((TQ,), jnp.float32),
            pltpu.VMEM((TQ,), jnp.float32),
            pltpu.VMEM((NUM_REG, D), jnp.float32),
            pltpu.VMEM((NUM_REG, D), jnp.float32),
            pltpu.SemaphoreType.DMA,
            pltpu.SemaphoreType.DMA,
        ],
        compiler_params=pltpu.CompilerParams(use_tc_tiling_on_sc=True,
                                             needs_layout_passes=False),
    )(_sc_body)
    return f(aiT, tblT, W, b)


def kernel(all_inputs, tables, W, b):
    aiT = all_inputs.T
    tblT = jnp.transpose(tables, (0, 2, 1))
    outT = _run(aiT, tblT, W, b)
    return jnp.transpose(outT, (2, 0, 1))
